# baseline (speedup 1.0000x reference)
import functools

import jax
import jax.numpy as jnp
from jax import lax
from jax.experimental import pallas as pl
from jax.experimental.pallas import tpu as pltpu
from jax.experimental.pallas import tpu_sc as plsc

B = 128
V = 100000
NW = 32
ROWS_PER_W = B // NW
LCH = 20000
NCH = 10000
NLC = V // LCH
NNC = V // NCH
L = 16


def _sampler_body(logits_hbm, temps_hbm, noise_hbm, out_hbm,
                  lv, nb0, nb1, tv, ov,
                  sl0, sl1, sl2, sl3, sl4, sn0, sn1):
    wid = lax.axis_index("c") * 16 + lax.axis_index("s")
    nbufs = (nb0, nb1)
    nsems = (sn0, sn1)
    lsems = (sl0, sl1, sl2, sl3, sl4)
    iota = lax.broadcasted_iota(jnp.int32, (L,), 0)

    def lane_take(v, idx):
        dnums = lax.GatherDimensionNumbers(
            offset_dims=(), collapsed_slice_dims=(0,), start_index_map=(0,))
        return lax.gather(
            v, idx[:, None], dnums, (1,),
            mode=lax.GatherScatterMode.PROMISE_IN_BOUNDS)

    def bcast_max(v):
        for sh in (8, 4, 2, 1):
            v = jnp.maximum(v, lane_take(v, (iota + sh) & (L - 1)))
        return v

    res = jnp.zeros((L,), jnp.int32)
    for r in range(ROWS_PER_W):
        row = wid * ROWS_PER_W + r
        lh = [
            pltpu.async_copy(
                logits_hbm.at[pl.ds(row * V + c * LCH, LCH)],
                lv.at[pl.ds(c * LCH, LCH)],
                lsems[c],
            )
            for c in range(NLC)
        ]
        nh = [
            pltpu.async_copy(
                noise_hbm.at[pl.ds(row * V + c * NCH, NCH)],
                nbufs[c], nsems[c])
            for c in range(2)
        ]
        pltpu.sync_copy(temps_hbm.at[pl.ds(row * L, L)], tv)

        mx = jnp.full((L,), -jnp.inf, jnp.float32)
        for c in range(NLC):
            lh[c].wait()

            def amax_body(j, m, base=c * LCH):
                return jnp.maximum(m, lv[pl.ds(base + j * L, L)])

            mx = lax.fori_loop(0, LCH // L, amax_body, mx)
        maxl = bcast_max(mx)

        t_raw = tv[...]
        gmask = t_raw == 0.0
        t_eff = jnp.where(gmask, jnp.ones((L,), jnp.float32), t_raw)
        m_splat = maxl / t_eff

        best_s = jnp.full((L,), -1.0, jnp.float32)
        best_i = jnp.zeros((L,), jnp.int32)
        for c in range(NNC):
            bi = c % 2
            nh[bi].wait()
            base = c * NCH
            nbuf = nbufs[bi]

            def score_body(j, carry, base=base, nbuf=nbuf,
                           t_eff=t_eff, m_splat=m_splat, gmask=gmask):
                bs, bix = carry
                off = base + j * L
                l = lv[pl.ds(off, L)]
                n = nbuf[pl.ds(j * L, L)]
                d = l / t_eff
                e = jnp.exp(d - m_splat)
                s = e / jnp.maximum(n, 1e-10)
                s = jnp.where(gmask, e, s)
                idxv = iota + jnp.full((L,), off, jnp.int32)
                gt = s > bs
                return jnp.where(gt, s, bs), jnp.where(gt, idxv, bix)

            best_s, best_i = lax.fori_loop(
                0, NCH // L, score_body, (best_s, best_i))
            nxt = c + 2
            if nxt < NNC:
                nh[bi] = pltpu.async_copy(
                    noise_hbm.at[pl.ds(row * V + nxt * NCH, NCH)],
                    nbufs[bi], nsems[bi])

        gmax = bcast_max(best_s)
        eq = best_s == gmax
        cand = jnp.where(eq, best_i, jnp.full((L,), 0x7FFFFFFF, jnp.int32))
        gidx = -bcast_max(-cand)
        res = jnp.where(iota == r, gidx, res)

    ov[...] = res
    pltpu.sync_copy(ov, out_hbm.at[pl.ds(wid * L, L)])


_sampler = functools.partial(
    pl.kernel,
    mesh=plsc.VectorSubcoreMesh(core_axis_name="c", subcore_axis_name="s"),
    out_type=jax.ShapeDtypeStruct((NW * L,), jnp.int32),
    scratch_types=[
        pltpu.VMEM((V,), jnp.float32),
        pltpu.VMEM((NCH,), jnp.float32),
        pltpu.VMEM((NCH,), jnp.float32),
        pltpu.VMEM((L,), jnp.float32),
        pltpu.VMEM((L,), jnp.int32),
    ] + [pltpu.SemaphoreType.DMA] * 7,
)(_sampler_body)


def kernel(logits, temperatures, exp_noise):
    temps_b = jnp.broadcast_to(
        temperatures.astype(jnp.float32)[:, None], (B, L)).reshape(B * L)
    out = _sampler(logits.astype(jnp.float32).reshape(B * V),
                   temps_b, exp_noise.reshape(B * V))
    return out.reshape(NW, L)[:, :ROWS_PER_W].reshape(B)
